# baseline (speedup 1.0000x reference)
import functools

import jax
import jax.numpy as jnp
from jax import lax
from jax.experimental import pallas as pl
from jax.experimental.pallas import tpu as pltpu
from jax.experimental.pallas import tpu_sc as plsc

N_NODES = 10000
N_EDGES = 320000
EMB = 160
H1 = 32
H1W = H1 // 2
H2 = 16

_INFO = plsc.get_sparse_core_info()
_NC, _NS, _NL = _INFO.num_cores, _INFO.num_subcores, _INFO.num_lanes
_NW = _NC * _NS
_CHUNK = 128
_NCHUNK = N_EDGES // _CHUNK


def _precompute_body(cust_ref, prod_ref, w1_ref, b1_ref, ab_ref):
    w_top = w1_ref[0:EMB, :]
    w_bot = w1_ref[EMB : 2 * EMB, :]
    ab_ref[0:N_NODES, :] = (
        jnp.dot(cust_ref[...], w_top, preferred_element_type=jnp.float32)
        + b1_ref[...]
    ).astype(jnp.bfloat16)
    ab_ref[N_NODES : 2 * N_NODES, :] = jnp.dot(
        prod_ref[...], w_bot, preferred_element_type=jnp.float32
    ).astype(jnp.bfloat16)


def _precompute(cust, prod, w1, b1):
    return pl.pallas_call(
        _precompute_body,
        out_shape=jax.ShapeDtypeStruct((2 * N_NODES, H1), jnp.bfloat16),
    )(cust, prod, w1, b1.reshape(1, H1))


_CPW = -(-_NCHUNK // _NW)
_DEPTH = 3


def _gather_body(ab_hbm, idx_hbm, out_hbm,
                 ab_sp, idx, rab, gt,
                 gs0, ws0, gs1, ws1, gs2, ws2):
    wid = lax.axis_index("s") * _NC + lax.axis_index("c")
    sid = lax.axis_index("s")
    base = jnp.minimum(wid * _CPW, _NCHUNK - _CPW)
    lane = lax.iota(jnp.int32, _NL)
    sems = ((gs0, ws0), (gs1, ws1), (gs2, ws2))

    @pl.when(sid == 0)
    def _():
        pltpu.sync_copy(ab_hbm, ab_sp)

    pltpu.sync_copy(idx_hbm.at[pl.ds(base, _CPW)], idx)
    plsc.subcore_barrier()
    for p in range(_DEPTH - 1):
        pltpu.async_copy(ab_sp.at[idx.at[p]], rab.at[p], sems[p][0])

    def group_body(j0, carry):
        for b in range(_DEPTH):
            j = j0 * _DEPTH + b
            bn = (b + _DEPTH - 1) % _DEPTH
            sg, sw = sems[b]
            ng, _ = sems[bn]

            @pl.when(j + _DEPTH - 1 < _CPW)
            def _():
                pltpu.async_copy(
                    ab_sp.at[idx.at[j + _DEPTH - 1]], rab.at[bn], ng)

            @pl.when(j < _CPW)
            def _():
                pltpu.make_async_copy(
                    ab_sp.at[idx.at[j]], rab.at[b], sg).wait()

                @pl.when(j >= _DEPTH)
                def _():
                    pltpu.make_async_copy(
                        gt.at[b], out_hbm.at[base + j - _DEPTH], sw).wait()

                rabv = rab.at[b]

                for jp in range(H1W):
                    jv = jnp.full((_NL,), jp, jnp.int32)
                    ais = []
                    bis = []
                    for g in range(_CHUNK // _NL):
                        rows = lane + g * _NL
                        ais.append(plsc.load_gather(rabv, [rows, jv]))
                        bis.append(
                            plsc.load_gather(rabv, [rows + _CHUNK, jv]))
                    for g in range(_CHUNK // _NL):
                        af = plsc.bitcast(ais[g], jnp.bfloat16)
                        bf = plsc.bitcast(bis[g], jnp.bfloat16)
                        h = jnp.maximum(af + bf, jnp.bfloat16(0))
                        gt[b, jp, pl.ds(g * _NL, _NL)] = plsc.bitcast(
                            h, jnp.int32)
                pltpu.async_copy(gt.at[b], out_hbm.at[base + j], sw)

        return carry

    lax.fori_loop(0, _CPW // _DEPTH + 1, group_body, 0)
    for b in range(_DEPTH):
        pltpu.make_async_copy(gt.at[b], out_hbm.at[base], sems[b][1]).wait()


def _gather_add_relu(ab_tab, idx2):
    mesh = plsc.VectorSubcoreMesh(core_axis_name="c", subcore_axis_name="s")
    f = functools.partial(
        pl.kernel,
        mesh=mesh,
        out_type=jax.ShapeDtypeStruct((_NCHUNK, H1W, _CHUNK), jnp.int32),
        compiler_params=pltpu.CompilerParams(
            use_tc_tiling_on_sc=False, needs_layout_passes=False
        ),
        scratch_types=[
            pltpu.VMEM_SHARED((2 * N_NODES, H1W), jnp.int32),
            pltpu.VMEM((_CPW, 2 * _CHUNK), jnp.int32),
            pltpu.VMEM((_DEPTH, 2 * _CHUNK, H1W), jnp.int32),
            pltpu.VMEM((_DEPTH, H1W, _CHUNK), jnp.int32),
        ] + [pltpu.SemaphoreType.DMA] * (2 * _DEPTH),
    )(_gather_body)
    return f(ab_tab, idx2)


_CB = 125


def _mlp_body(g_ref, w2_ref, b2_ref, w3_ref, b3_ref, out_ref):
    gi = g_ref[...]
    gw = jnp.concatenate([gi[k] for k in range(_CB)], axis=1)
    gb = pltpu.bitcast(gw, jnp.bfloat16)
    h = lax.dot_general(
        w2_ref[...], gb, (((0,), (0,)), ((), ())),
        preferred_element_type=jnp.float32,
    )
    h = jnp.maximum(h + b2_ref[...].reshape(H2, 1), 0.0)
    o = lax.dot_general(
        w3_ref[...], h, (((0,), (0,)), ((), ())),
        preferred_element_type=jnp.float32,
    ) + b3_ref[...]
    i = pl.program_id(0)
    out_ref[pl.ds(i * _CB, _CB), :] = jax.nn.sigmoid(o).reshape(_CB, _CHUNK)


def _mlp_tail(g3, w2, b2, w3, b3):
    grid = _NCHUNK // _CB
    return pl.pallas_call(
        _mlp_body,
        grid=(grid,),
        in_specs=[
            pl.BlockSpec((_CB, H1W, _CHUNK), lambda i: (i, 0, 0)),
            pl.BlockSpec((H1, H2), lambda i: (0, 0)),
            pl.BlockSpec((1, H2), lambda i: (0, 0)),
            pl.BlockSpec((H2, 1), lambda i: (0, 0)),
            pl.BlockSpec((1, 1), lambda i: (0, 0)),
        ],
        out_specs=pl.BlockSpec((_NCHUNK, _CHUNK), lambda i: (0, 0)),
        out_shape=jax.ShapeDtypeStruct((_NCHUNK, _CHUNK), jnp.float32),
    )(g3, w2, b2.reshape(1, H2), w3, b3.reshape(1, 1))


def kernel(customer_emb, product_emb, edge_index, W1, b1, W2, b2, W3, b3):
    ab_bf = _precompute(customer_emb, product_emb, W1, b1)
    ab_i32 = lax.bitcast_convert_type(
        ab_bf.reshape(2 * N_NODES, H1W, 2), jnp.int32)
    row2d = edge_index[0].reshape(_NCHUNK, _CHUNK)
    col2d = edge_index[1].reshape(_NCHUNK, _CHUNK) + N_NODES
    idx2 = jnp.concatenate([row2d, col2d], axis=1)
    g3 = _gather_add_relu(ab_i32, idx2)
    out2d = _mlp_tail(g3, W2.astype(jnp.bfloat16), b2, W3, b3)
    return out2d.reshape(N_EDGES)


# speedup vs baseline: 1.0226x; 1.0226x over previous
import functools

import jax
import jax.numpy as jnp
from jax import lax
from jax.experimental import pallas as pl
from jax.experimental.pallas import tpu as pltpu
from jax.experimental.pallas import tpu_sc as plsc

N_NODES = 10000
N_EDGES = 320000
EMB = 160
H1 = 32
H1W = H1 // 2
H2 = 16

_INFO = plsc.get_sparse_core_info()
_NC, _NS, _NL = _INFO.num_cores, _INFO.num_subcores, _INFO.num_lanes
_NW = _NC * _NS
_CHUNK = 128
_NCHUNK = N_EDGES // _CHUNK


def _precompute_body(cust_ref, prod_ref, w1_ref, b1_ref, ab_ref):
    w_top = w1_ref[0:EMB, :]
    w_bot = w1_ref[EMB : 2 * EMB, :]
    ab_ref[0:N_NODES, :] = (
        jnp.dot(cust_ref[...], w_top, preferred_element_type=jnp.float32)
        + b1_ref[...]
    ).astype(jnp.bfloat16)
    ab_ref[N_NODES : 2 * N_NODES, :] = jnp.dot(
        prod_ref[...], w_bot, preferred_element_type=jnp.float32
    ).astype(jnp.bfloat16)


def _precompute(cust, prod, w1, b1):
    return pl.pallas_call(
        _precompute_body,
        out_shape=jax.ShapeDtypeStruct((2 * N_NODES, H1), jnp.bfloat16),
    )(cust, prod, w1, b1.reshape(1, H1))


_CPW = -(-_NCHUNK // _NW)
_DEPTH = 3


def _gather_body(ab_hbm, idx_hbm, out_hbm,
                 ab_sp, idx, rab, gt,
                 ga0, gb0, ws0, ga1, gb1, ws1, ga2, gb2, ws2):
    wid = lax.axis_index("s") * _NC + lax.axis_index("c")
    sid = lax.axis_index("s")
    base = jnp.minimum(wid * _CPW, _NCHUNK - _CPW)
    lane = lax.iota(jnp.int32, _NL)
    sems = ((ga0, gb0, ws0), (ga1, gb1, ws1), (ga2, gb2, ws2))

    def issue_gather(p, slot):
        pltpu.async_copy(
            ab_sp.at[idx.at[p, pl.ds(0, _CHUNK)]],
            rab.at[slot, pl.ds(0, _CHUNK)], sems[slot][0])
        pltpu.async_copy(
            ab_sp.at[idx.at[p, pl.ds(_CHUNK, _CHUNK)]],
            rab.at[slot, pl.ds(_CHUNK, _CHUNK)], sems[slot][1])

    def wait_gather(p, slot):
        pltpu.make_async_copy(
            ab_sp.at[idx.at[p, pl.ds(0, _CHUNK)]],
            rab.at[slot, pl.ds(0, _CHUNK)], sems[slot][0]).wait()
        pltpu.make_async_copy(
            ab_sp.at[idx.at[p, pl.ds(_CHUNK, _CHUNK)]],
            rab.at[slot, pl.ds(_CHUNK, _CHUNK)], sems[slot][1]).wait()

    @pl.when(sid == 0)
    def _():
        pltpu.sync_copy(ab_hbm, ab_sp)

    pltpu.sync_copy(idx_hbm.at[pl.ds(base, _CPW)], idx)
    plsc.subcore_barrier()
    for p in range(_DEPTH - 1):
        issue_gather(p, p)

    def group_body(j0, carry):
        for b in range(_DEPTH):
            j = j0 * _DEPTH + b
            bn = (b + _DEPTH - 1) % _DEPTH
            sw = sems[b][2]

            @pl.when(j + _DEPTH - 1 < _CPW)
            def _():
                issue_gather(j + _DEPTH - 1, bn)

            @pl.when(j < _CPW)
            def _():
                wait_gather(j, b)

                @pl.when(j >= _DEPTH)
                def _():
                    pltpu.make_async_copy(
                        gt.at[b], out_hbm.at[base + j - _DEPTH], sw).wait()

                rabv = rab.at[b]

                for jp in range(H1W):
                    jv = jnp.full((_NL,), jp, jnp.int32)
                    ais = []
                    bis = []
                    for g in range(_CHUNK // _NL):
                        rows = lane + g * _NL
                        ais.append(plsc.load_gather(rabv, [rows, jv]))
                        bis.append(
                            plsc.load_gather(rabv, [rows + _CHUNK, jv]))
                    for g in range(_CHUNK // _NL):
                        af = plsc.bitcast(ais[g], jnp.bfloat16)
                        bf = plsc.bitcast(bis[g], jnp.bfloat16)
                        h = jnp.maximum(af + bf, jnp.bfloat16(0))
                        gt[b, jp, pl.ds(g * _NL, _NL)] = plsc.bitcast(
                            h, jnp.int32)
                pltpu.async_copy(gt.at[b], out_hbm.at[base + j], sw)

        return carry

    lax.fori_loop(0, _CPW // _DEPTH + 1, group_body, 0)
    for b in range(_DEPTH):
        pltpu.make_async_copy(gt.at[b], out_hbm.at[base], sems[b][2]).wait()


def _gather_add_relu(ab_tab, idx2):
    mesh = plsc.VectorSubcoreMesh(core_axis_name="c", subcore_axis_name="s")
    f = functools.partial(
        pl.kernel,
        mesh=mesh,
        out_type=jax.ShapeDtypeStruct((_NCHUNK, H1W, _CHUNK), jnp.int32),
        compiler_params=pltpu.CompilerParams(
            use_tc_tiling_on_sc=False, needs_layout_passes=False
        ),
        scratch_types=[
            pltpu.VMEM_SHARED((2 * N_NODES, H1W), jnp.int32),
            pltpu.VMEM((_CPW, 2 * _CHUNK), jnp.int32),
            pltpu.VMEM((_DEPTH, 2 * _CHUNK, H1W), jnp.int32),
            pltpu.VMEM((_DEPTH, H1W, _CHUNK), jnp.int32),
        ] + [pltpu.SemaphoreType.DMA] * (3 * _DEPTH),
    )(_gather_body)
    return f(ab_tab, idx2)


_CB = 125


def _mlp_body(g_ref, w2_ref, b2_ref, w3_ref, b3_ref, out_ref):
    gi = g_ref[...]
    gw = jnp.concatenate([gi[k] for k in range(_CB)], axis=1)
    gb = pltpu.bitcast(gw, jnp.bfloat16)
    h = lax.dot_general(
        w2_ref[...], gb, (((0,), (0,)), ((), ())),
        preferred_element_type=jnp.float32,
    )
    h = jnp.maximum(h + b2_ref[...].reshape(H2, 1), 0.0)
    o = lax.dot_general(
        w3_ref[...], h, (((0,), (0,)), ((), ())),
        preferred_element_type=jnp.float32,
    ) + b3_ref[...]
    i = pl.program_id(0)
    out_ref[pl.ds(i * _CB, _CB), :] = jax.nn.sigmoid(o).reshape(_CB, _CHUNK)


def _mlp_tail(g3, w2, b2, w3, b3):
    grid = _NCHUNK // _CB
    return pl.pallas_call(
        _mlp_body,
        grid=(grid,),
        in_specs=[
            pl.BlockSpec((_CB, H1W, _CHUNK), lambda i: (i, 0, 0)),
            pl.BlockSpec((H1, H2), lambda i: (0, 0)),
            pl.BlockSpec((1, H2), lambda i: (0, 0)),
            pl.BlockSpec((H2, 1), lambda i: (0, 0)),
            pl.BlockSpec((1, 1), lambda i: (0, 0)),
        ],
        out_specs=pl.BlockSpec((_NCHUNK, _CHUNK), lambda i: (0, 0)),
        out_shape=jax.ShapeDtypeStruct((_NCHUNK, _CHUNK), jnp.float32),
    )(g3, w2, b2.reshape(1, H2), w3, b3.reshape(1, 1))


def kernel(customer_emb, product_emb, edge_index, W1, b1, W2, b2, W3, b3):
    ab_bf = _precompute(customer_emb, product_emb, W1, b1)
    ab_i32 = lax.bitcast_convert_type(
        ab_bf.reshape(2 * N_NODES, H1W, 2), jnp.int32)
    row2d = edge_index[0].reshape(_NCHUNK, _CHUNK)
    col2d = edge_index[1].reshape(_NCHUNK, _CHUNK) + N_NODES
    idx2 = jnp.concatenate([row2d, col2d], axis=1)
    g3 = _gather_add_relu(ab_i32, idx2)
    out2d = _mlp_tail(g3, W2.astype(jnp.bfloat16), b2, W3, b3)
    return out2d.reshape(N_EDGES)


# speedup vs baseline: 1.0560x; 1.0326x over previous
import functools

import jax
import jax.numpy as jnp
from jax import lax
from jax.experimental import pallas as pl
from jax.experimental.pallas import tpu as pltpu
from jax.experimental.pallas import tpu_sc as plsc

N_NODES = 10000
N_EDGES = 320000
EMB = 160
H1 = 32
H1W = H1 // 2
H2 = 16

_INFO = plsc.get_sparse_core_info()
_NC, _NS, _NL = _INFO.num_cores, _INFO.num_subcores, _INFO.num_lanes
_NW = _NC * _NS
_CHUNK = 128
_NCHUNK = N_EDGES // _CHUNK


def _precompute_body(cust_ref, prod_ref, w1_ref, b1_ref, a_ref, b_ref):
    w_top = w1_ref[0:EMB, :]
    w_bot = w1_ref[EMB : 2 * EMB, :]
    a_ref[...] = (
        jnp.dot(cust_ref[...], w_top, preferred_element_type=jnp.float32)
        + b1_ref[...]
    ).astype(jnp.bfloat16)
    b_ref[...] = jnp.dot(
        prod_ref[...], w_bot, preferred_element_type=jnp.float32
    ).astype(jnp.bfloat16)


def _precompute(cust, prod, w1, b1):
    return pl.pallas_call(
        _precompute_body,
        out_shape=(
            jax.ShapeDtypeStruct((N_NODES, H1), jnp.bfloat16),
            jax.ShapeDtypeStruct((N_NODES, H1), jnp.bfloat16),
        ),
    )(cust, prod, w1, b1.reshape(1, H1))


_CPW = -(-_NCHUNK // _NW)
_PADCHUNK = _CPW * _NW
_DEPTH = 3


def _gather_body(a_hbm, b_hbm, row_hbm, col_hbm, out_hbm,
                 a_sp, b_sp, idxr, idxc, ra, rb, gt,
                 gsa0, gsb0, ws0, gsa1, gsb1, ws1, gsa2, gsb2, ws2):
    wid = lax.axis_index("s") * _NC + lax.axis_index("c")
    sid = lax.axis_index("s")
    base = wid * _CPW
    cnt = jnp.minimum(_CPW, _NCHUNK - base)
    lane = lax.iota(jnp.int32, _NL)
    sems = ((gsa0, gsb0, ws0), (gsa1, gsb1, ws1), (gsa2, gsb2, ws2))

    @pl.when(sid == 0)
    def _():
        pltpu.sync_copy(a_hbm, a_sp)
        pltpu.sync_copy(b_hbm, b_sp)

    pltpu.sync_copy(row_hbm.at[pl.ds(base, _CPW)], idxr)
    pltpu.sync_copy(col_hbm.at[pl.ds(base, _CPW)], idxc)
    plsc.subcore_barrier()
    for p in range(_DEPTH - 1):
        pltpu.async_copy(a_sp.at[idxr.at[p]], ra.at[p], sems[p][0])
        pltpu.async_copy(b_sp.at[idxc.at[p]], rb.at[p], sems[p][1])

    def group_body(j0, carry):
        for b in range(_DEPTH):
            j = j0 * _DEPTH + b
            bn = (b + _DEPTH - 1) % _DEPTH
            sa, sb, sw = sems[b]
            na, nb_, _ = sems[bn]

            @pl.when(j + _DEPTH - 1 < cnt)
            def _():
                pltpu.async_copy(
                    a_sp.at[idxr.at[j + _DEPTH - 1]], ra.at[bn], na)
                pltpu.async_copy(
                    b_sp.at[idxc.at[j + _DEPTH - 1]], rb.at[bn], nb_)

            @pl.when(j < cnt)
            def _():
                pltpu.make_async_copy(a_sp.at[idxr.at[j]], ra.at[b], sa).wait()
                pltpu.make_async_copy(b_sp.at[idxc.at[j]], rb.at[b], sb).wait()

                @pl.when(j >= _DEPTH)
                def _():
                    pltpu.make_async_copy(
                        gt.at[b], out_hbm.at[base + j - _DEPTH], sw).wait()

                rav = ra.at[b]
                rbv = rb.at[b]

                for jp in range(H1W):
                    jv = jnp.full((_NL,), jp, jnp.int32)
                    ais = []
                    bis = []
                    for g in range(_CHUNK // _NL):
                        rows = lane + g * _NL
                        ais.append(plsc.load_gather(rav, [rows, jv]))
                        bis.append(plsc.load_gather(rbv, [rows, jv]))
                    for g in range(_CHUNK // _NL):
                        af = plsc.bitcast(ais[g], jnp.bfloat16)
                        bf = plsc.bitcast(bis[g], jnp.bfloat16)
                        gt[b, jp, pl.ds(g * _NL, _NL)] = plsc.bitcast(
                            af + bf, jnp.int32)
                pltpu.async_copy(gt.at[b], out_hbm.at[base + j], sw)

        return carry

    lax.fori_loop(0, _CPW // _DEPTH + 1, group_body, 0)
    for b in range(_DEPTH):
        pltpu.make_async_copy(gt.at[b], out_hbm.at[base], sems[b][2]).wait()


def _gather_add(a_tab, b_tab, row2d, col2d):
    mesh = plsc.VectorSubcoreMesh(core_axis_name="c", subcore_axis_name="s")
    f = functools.partial(
        pl.kernel,
        mesh=mesh,
        out_type=jax.ShapeDtypeStruct((_NCHUNK, H1W, _CHUNK), jnp.int32),
        compiler_params=pltpu.CompilerParams(
            use_tc_tiling_on_sc=False, needs_layout_passes=False
        ),
        scratch_types=[
            pltpu.VMEM_SHARED((N_NODES, H1W), jnp.int32),
            pltpu.VMEM_SHARED((N_NODES, H1W), jnp.int32),
            pltpu.VMEM((_CPW, _CHUNK), jnp.int32),
            pltpu.VMEM((_CPW, _CHUNK), jnp.int32),
            pltpu.VMEM((_DEPTH, _CHUNK, H1W), jnp.int32),
            pltpu.VMEM((_DEPTH, _CHUNK, H1W), jnp.int32),
            pltpu.VMEM((_DEPTH, H1W, _CHUNK), jnp.int32),
        ] + [pltpu.SemaphoreType.DMA] * (3 * _DEPTH),
    )(_gather_body)
    return f(a_tab, b_tab, row2d, col2d)


_CB = 125


def _mlp_body(g_ref, w2_ref, b2_ref, w3_ref, b3_ref, out_ref):
    gi = g_ref[...]
    gw = jnp.concatenate([gi[k] for k in range(_CB)], axis=1)
    gb = jnp.maximum(pltpu.bitcast(gw, jnp.bfloat16), jnp.bfloat16(0))
    h = lax.dot_general(
        w2_ref[...], gb, (((0,), (0,)), ((), ())),
        preferred_element_type=jnp.float32,
    )
    h = jnp.maximum(h + b2_ref[...].reshape(H2, 1), 0.0)
    o = lax.dot_general(
        w3_ref[...], h, (((0,), (0,)), ((), ())),
        preferred_element_type=jnp.float32,
    ) + b3_ref[...]
    i = pl.program_id(0)
    out_ref[pl.ds(i * _CB, _CB), :] = jax.nn.sigmoid(o).reshape(_CB, _CHUNK)


def _mlp_tail(g3, w2, b2, w3, b3):
    grid = _NCHUNK // _CB
    return pl.pallas_call(
        _mlp_body,
        grid=(grid,),
        in_specs=[
            pl.BlockSpec((_CB, H1W, _CHUNK), lambda i: (i, 0, 0)),
            pl.BlockSpec((H1, H2), lambda i: (0, 0)),
            pl.BlockSpec((1, H2), lambda i: (0, 0)),
            pl.BlockSpec((H2, 1), lambda i: (0, 0)),
            pl.BlockSpec((1, 1), lambda i: (0, 0)),
        ],
        out_specs=pl.BlockSpec((_NCHUNK, _CHUNK), lambda i: (0, 0)),
        out_shape=jax.ShapeDtypeStruct((_NCHUNK, _CHUNK), jnp.float32),
    )(g3, w2, b2.reshape(1, H2), w3, b3.reshape(1, 1))


def kernel(customer_emb, product_emb, edge_index, W1, b1, W2, b2, W3, b3):
    a_bf, b_bf = _precompute(customer_emb, product_emb, W1, b1)
    a_i32 = lax.bitcast_convert_type(
        a_bf.reshape(N_NODES, H1W, 2), jnp.int32)
    b_i32 = lax.bitcast_convert_type(
        b_bf.reshape(N_NODES, H1W, 2), jnp.int32)
    pad = ((0, _PADCHUNK - _NCHUNK), (0, 0))
    row2d = jnp.pad(edge_index[0].reshape(_NCHUNK, _CHUNK), pad)
    col2d = jnp.pad(edge_index[1].reshape(_NCHUNK, _CHUNK), pad)
    g3 = _gather_add(a_i32, b_i32, row2d, col2d)
    out2d = _mlp_tail(g3, W2.astype(jnp.bfloat16), b2, W3, b3)
    return out2d.reshape(N_EDGES)
